# Initial kernel scaffold; baseline (speedup 1.0000x reference)
import jax
import jax.numpy as jnp
from jax.experimental import pallas as pl


def kernel(x, edge_index, nfc_w, nfc_b, efc_w, efc_b, ln_g, ln_b, betas, w1, b1, g1, bb1, w2, b2, phi_w, phi_b, wa, ba, wb, bb_attn, wc, bc, rho_w, rho_b, cls_w, cls_b):
    raise NotImplementedError("write your pallas kernel here")


# speedup vs baseline: 6.6795x; 6.6795x over previous
import functools

import jax
import jax.numpy as jnp
from jax import lax
from jax.experimental import pallas as pl
from jax.experimental.pallas import tpu as pltpu
from jax.experimental.pallas import tpu_sc as plsc

N = 50000
E = 800000
H = 64
L = 3
EPS = 1e-07

BN = 2000
NB = N // BN

_NS = 16
LANE = 128
RPT = 400
E_ROWS = RPT * _NS
E_PAD = E_ROWS * LANE
IB = 16
OB = RPT // IB
FPT = 3128
NF = FPT * _NS
NACC = NF


def _ln(x, g, b, eps=1e-5):
    mu = jnp.mean(x, axis=-1, keepdims=True)
    var = jnp.var(x, axis=-1, keepdims=True)
    return (x - mu) * jax.lax.rsqrt(var + eps) * g + b


def _prep_tables(hv, lng, lnb, efcw, efcb, beta):
    hvn = jax.nn.relu(_ln(hv, lng, lnb))
    c = jax.nn.relu(efcw + efcb)
    p = jax.nn.relu(hvn + c) + EPS
    w = jnp.exp(beta * p)
    u = p * w
    return hvn, u, w


def _write_T(T_ref, u, w):
    T_ref[0, :, :] = u[:, 0:32]
    T_ref[1, :, :] = u[:, 32:64]
    T_ref[2, :, :] = w[:, 0:32]
    T_ref[3, :, :] = w[:, 32:64]


def _k0_body(x_ref, nfcw_ref, nfcb_ref, efcw_ref, efcb_ref, lng_ref, lnb_ref,
             beta_ref, hv0_ref, hvn_ref, T_ref):
    hv = jnp.dot(x_ref[...], nfcw_ref[...], preferred_element_type=jnp.float32)
    hv = jax.nn.relu(hv + nfcb_ref[...])
    hv0_ref[...] = hv
    hvn, u, w = _prep_tables(hv, lng_ref[...], lnb_ref[...], efcw_ref[...],
                             efcb_ref[...], beta_ref[...])
    hvn_ref[...] = hvn
    _write_T(T_ref, u, w)


def _k0(x, nfcw, nfcb, efcw, efcb, lng, lnb, beta):
    full = lambda shp: pl.BlockSpec(shp, lambda i: (0,) * len(shp))
    return pl.pallas_call(
        _k0_body,
        grid=(NB,),
        in_specs=[
            pl.BlockSpec((BN, 256), lambda i: (i, 0)),
            full((256, H)), full((1, H)), full((1, H)), full((1, H)),
            full((1, H)), full((1, H)), full((1, 1)),
        ],
        out_specs=[
            pl.BlockSpec((BN, H), lambda i: (i, 0)),
            pl.BlockSpec((BN, H), lambda i: (i, 0)),
            pl.BlockSpec((4, BN, 32), lambda i: (0, i, 0)),
        ],
        out_shape=[
            jax.ShapeDtypeStruct((N, H), jnp.float32),
            jax.ShapeDtypeStruct((N, H), jnp.float32),
            jax.ShapeDtypeStruct((4, N, 32), jnp.float32),
        ],
    )(x, nfcw, nfcb, efcw, efcb, lng, lnb, beta)


def _klayer_body(prep, hvn_ref, acc_ref, hvp_ref, w1_ref, b1_ref, g1_ref,
                 bb1_ref, w2_ref, b2_ref, *rest):
    if prep:
        (efcw_ref, efcb_ref, lng_ref, lnb_ref, beta_ref,
         hv_ref, hvn2_ref, T_ref) = rest
    else:
        (hv_ref,) = rest
    numer = jnp.concatenate([acc_ref[0, :, :], acc_ref[1, :, :]], axis=-1)
    denom = jnp.concatenate([acc_ref[2, :, :], acc_ref[3, :, :]], axis=-1)
    good = denom > 0
    msg = jnp.where(good, numer / jnp.where(good, denom, 1.0), 0.0)
    feats = hvn_ref[...] + msg
    h = jnp.dot(feats, w1_ref[...], preferred_element_type=jnp.float32)
    h = jax.nn.relu(_ln(h + b1_ref[...], g1_ref[...], bb1_ref[...]))
    hv = jnp.dot(h, w2_ref[...], preferred_element_type=jnp.float32)
    hv = hv + b2_ref[...] + hvp_ref[...]
    hv_ref[...] = hv
    if prep:
        hvn, u, w = _prep_tables(hv, lng_ref[...], lnb_ref[...], efcw_ref[...],
                                 efcb_ref[...], beta_ref[...])
        hvn2_ref[...] = hvn
        _write_T(T_ref, u, w)


def _klayer(prep, hvn, acc, hvp, w1, b1, g1, bb1, w2, b2, *extra):
    full = lambda shp: pl.BlockSpec(shp, lambda i: (0,) * len(shp))
    rowspec = pl.BlockSpec((BN, H), lambda i: (i, 0))
    in_specs = [
        rowspec,
        pl.BlockSpec((4, BN, 32), lambda i: (0, i, 0)),
        rowspec,
        full((H, 2 * H)), full((1, 2 * H)), full((1, 2 * H)),
        full((1, 2 * H)), full((2 * H, H)), full((1, H)),
    ]
    out_specs = [rowspec]
    out_shape = [jax.ShapeDtypeStruct((N, H), jnp.float32)]
    if prep:
        in_specs += [full((1, H)), full((1, H)), full((1, H)), full((1, H)),
                     full((1, 1))]
        out_specs += [rowspec, pl.BlockSpec((4, BN, 32), lambda i: (0, i, 0))]
        out_shape += [jax.ShapeDtypeStruct((N, H), jnp.float32),
                      jax.ShapeDtypeStruct((4, N, 32), jnp.float32)]
    return pl.pallas_call(
        functools.partial(_klayer_body, prep),
        grid=(NB,),
        in_specs=in_specs,
        out_specs=out_specs,
        out_shape=out_shape,
    )(hvn, acc, hvp, w1, b1, g1, bb1, w2, b2, *extra)


def _k4_body(hv0_ref, hv1_ref, hv2_ref, hv3_ref, phiw_ref, phib_ref, wa_ref,
             ba_ref, wb_ref, bbat_ref, wc_ref, bc_ref, rhow_ref, rhob_ref,
             clsw_ref, clsb_ref, hpath_ref, out_ref, outfeat_ref,
             acch_ref, acce_ref):
    i = pl.program_id(0)
    xcat = jnp.concatenate(
        [hv0_ref[...], hv1_ref[...], hv2_ref[...], hv3_ref[...]], axis=-1)
    hp = jnp.dot(xcat, phiw_ref[...], preferred_element_type=jnp.float32)
    hp = jax.nn.relu(hp + phib_ref[...])
    hpath_ref[...] = hp
    a = jnp.tanh(jnp.dot(hp, wa_ref[...], preferred_element_type=jnp.float32)
                 + ba_ref[...])
    b = jax.nn.sigmoid(
        jnp.dot(hp, wb_ref[...], preferred_element_type=jnp.float32)
        + bbat_ref[...])
    gate = jnp.dot(a * b, wc_ref[...], preferred_element_type=jnp.float32)
    gate = gate + bc_ref[...]
    shift = jnp.sum(jnp.abs(wc_ref[...])) + jnp.abs(bc_ref[0, 0])
    e = jnp.exp(gate - shift)
    se = jnp.sum(e)
    seh = jnp.sum(e * hp, axis=0, keepdims=True)

    @pl.when(i == 0)
    def _():
        acch_ref[...] = seh
        acce_ref[...] = jnp.full((1, 256), se, jnp.float32)

    @pl.when(i > 0)
    def _():
        acch_ref[...] += seh
        acce_ref[...] += jnp.full((1, 256), se, jnp.float32)

    @pl.when(i == NB - 1)
    def _():
        hg = acch_ref[...] / acce_ref[...]
        of = jnp.dot(hg, rhow_ref[...], preferred_element_type=jnp.float32)
        of = jax.nn.relu(of + rhob_ref[...])
        outfeat_ref[...] = of
        out_ref[...] = jnp.dot(of, clsw_ref[...],
                               preferred_element_type=jnp.float32) + clsb_ref[...]


def _k4(hv0, hv1, hv2, hv3, phiw, phib, wa, ba, wb, bbat, wc, bc, rhow, rhob,
        clsw, clsb):
    full = lambda shp: pl.BlockSpec(shp, lambda i: (0,) * len(shp))
    rowspec = pl.BlockSpec((BN, H), lambda i: (i, 0))
    D = 4 * H
    return pl.pallas_call(
        _k4_body,
        grid=(NB,),
        in_specs=[
            rowspec, rowspec, rowspec, rowspec,
            full((D, D)), full((1, D)), full((D, D)), full((1, D)),
            full((D, D)), full((1, D)), full((D, 1)), full((1, 1)),
            full((D, H)), full((1, H)), full((H, 2)), full((1, 2)),
        ],
        out_specs=[
            pl.BlockSpec((BN, D), lambda i: (i, 0)),
            full((1, 2)), full((1, H)),
        ],
        out_shape=[
            jax.ShapeDtypeStruct((N, D), jnp.float32),
            jax.ShapeDtypeStruct((1, 2), jnp.float32),
            jax.ShapeDtypeStruct((1, H), jnp.float32),
        ],
        scratch_shapes=[
            pltpu.VMEM((1, D), jnp.float32),
            pltpu.VMEM((1, D), jnp.float32),
        ],
    )(hv0, hv1, hv2, hv3, phiw, phib, wa, ba, wb, bbat, wc, bc, rhow, rhob,
      clsw, clsb)


def _sc_body(T_ref, src_ref, dst_ref, zr_ref, out_ref,
             sacc, idxs, idxd, rows, zbuf, gsem):
    c = lax.axis_index("c")
    s = lax.axis_index("s")
    pltpu.sync_copy(zr_ref, zbuf)

    fl0 = s * FPT

    def zero_own_range():
        def zstep(j, _):
            pltpu.sync_copy(zbuf, sacc.at[pl.ds(fl0 + j * 128, 128)])
            return 0
        lax.fori_loop(0, FPT // 128, zstep, 0)
        pltpu.sync_copy(zbuf.at[pl.ds(0, FPT - (FPT // 128) * 128)],
                        sacc.at[pl.ds(fl0 + (FPT // 128) * 128,
                                      FPT - (FPT // 128) * 128)])

    r_base = s * RPT
    for sl in range(2):
        slab = 2 * c + sl
        zero_own_range()
        plsc.subcore_barrier()

        def estep(ob, _):
            r0 = r_base + ob * IB
            pltpu.sync_copy(src_ref.at[slab, pl.ds(r0, IB)], idxs)
            pltpu.sync_copy(dst_ref.at[pl.ds(r0, IB)], idxd)
            for jj in range(IB):
                pltpu.async_copy(T_ref.at[idxs.at[jj]], rows, gsem).wait()
                pltpu.sync_copy(rows, sacc.at[idxd.at[jj]], add=True)
            return 0
        lax.fori_loop(0, OB, estep, 0)

        plsc.subcore_barrier()
        pltpu.sync_copy(sacc.at[pl.ds(fl0, FPT)],
                        out_ref.at[pl.ds(slab * NF + fl0, FPT)])


@functools.partial(
    pl.kernel,
    out_type=jax.ShapeDtypeStruct((4 * NF, 32), jnp.float32),
    mesh=plsc.VectorSubcoreMesh(core_axis_name="c", subcore_axis_name="s"),
    scratch_types=[
        pltpu.VMEM_SHARED((NACC, 32), jnp.float32),
        pltpu.VMEM((IB, LANE), jnp.int32),
        pltpu.VMEM((IB, LANE), jnp.int32),
        pltpu.VMEM((LANE, 32), jnp.float32),
        pltpu.VMEM((LANE, 32), jnp.float32),
        pltpu.SemaphoreType.DMA,
    ],
    compiler_params=pltpu.CompilerParams(use_tc_tiling_on_sc=False),
)
def _sc_scatter(T_ref, src_ref, dst_ref, zr_ref, out_ref,
                sacc, idxs, idxd, rows, zbuf, gsem):
    _sc_body(T_ref, src_ref, dst_ref, zr_ref, out_ref,
             sacc, idxs, idxd, rows, zbuf, gsem)


def kernel(x, edge_index, nfc_w, nfc_b, efc_w, efc_b, ln_g, ln_b, betas,
           w1, b1, g1, bb1, w2, b2, phi_w, phi_b, wa, ba, wb, bb_attn,
           wc, bc, rho_w, rho_b, cls_w, cls_b):
    f32 = jnp.float32
    src = edge_index[0].astype(jnp.int32)
    dst = edge_index[1].astype(jnp.int32)
    pad = E_PAD - E
    srcp = jnp.concatenate([src, jnp.zeros((pad,), jnp.int32)])
    dstp = jnp.concatenate([dst, jnp.full((pad,), N, jnp.int32)])
    src4 = jnp.stack([srcp + s * N for s in range(4)]).reshape(4, E_ROWS, LANE)
    dst2 = dstp.reshape(E_ROWS, LANE)
    zrows = jnp.zeros((LANE, 32), f32)

    r2 = lambda v: v.reshape(1, -1)
    hv0, hvn, T = _k0(x, nfc_w, r2(nfc_b), efc_w, r2(efc_b),
                      r2(ln_g[0]), r2(ln_b[0]), betas[0].reshape(1, 1))
    hvs = [hv0]
    for l in range(L):
        acc = _sc_scatter(T.reshape(4 * N, 32), src4, dst2, zrows)
        acc = acc.reshape(4, NF, 32)
        prep = l < L - 1
        extra = ()
        if prep:
            extra = (efc_w, r2(efc_b), r2(ln_g[l + 1]), r2(ln_b[l + 1]),
                     betas[l + 1].reshape(1, 1))
        res = _klayer(prep, hvn, acc, hvs[-1], w1[l], r2(b1[l]), r2(g1[l]),
                      r2(bb1[l]), w2[l], r2(b2[l]), *extra)
        if prep:
            hv, hvn, T = res
        else:
            (hv,) = res
        hvs.append(hv)

    h_path, out, out_feat = _k4(hvs[0], hvs[1], hvs[2], hvs[3], phi_w,
                                r2(phi_b), wa, r2(ba), wb, r2(bb_attn), wc,
                                bc.reshape(1, 1), rho_w, r2(rho_b), cls_w,
                                r2(cls_b))
    return (out, out_feat, h_path)


# speedup vs baseline: 8.7285x; 1.3068x over previous
import functools

import jax
import jax.numpy as jnp
from jax import lax
from jax.experimental import pallas as pl
from jax.experimental.pallas import tpu as pltpu
from jax.experimental.pallas import tpu_sc as plsc

N = 50000
E = 800000
H = 64
L = 3
EPS = 1e-07

BN = 2000
NB = N // BN

_NS = 16
LANE = 128
RPT = 400
E_ROWS = RPT * _NS
E_PAD = E_ROWS * LANE
NR = 2
NBLK = RPT // NR
FPT = 3128
NF = FPT * _NS
NACC = NF


def _ln(x, g, b, eps=1e-5):
    mu = jnp.mean(x, axis=-1, keepdims=True)
    var = jnp.var(x, axis=-1, keepdims=True)
    return (x - mu) * jax.lax.rsqrt(var + eps) * g + b


def _prep_tables(hv, lng, lnb, efcw, efcb, beta):
    hvn = jax.nn.relu(_ln(hv, lng, lnb))
    c = jax.nn.relu(efcw + efcb)
    p = jax.nn.relu(hvn + c) + EPS
    w = jnp.exp(beta * p)
    u = p * w
    return hvn, u, w


def _write_T(T_ref, u, w):
    T_ref[0, :, :] = u[:, 0:32]
    T_ref[1, :, :] = u[:, 32:64]
    T_ref[2, :, :] = w[:, 0:32]
    T_ref[3, :, :] = w[:, 32:64]


def _k0_body(x_ref, nfcw_ref, nfcb_ref, efcw_ref, efcb_ref, lng_ref, lnb_ref,
             beta_ref, hv0_ref, hvn_ref, T_ref):
    hv = jnp.dot(x_ref[...], nfcw_ref[...], preferred_element_type=jnp.float32)
    hv = jax.nn.relu(hv + nfcb_ref[...])
    hv0_ref[...] = hv
    hvn, u, w = _prep_tables(hv, lng_ref[...], lnb_ref[...], efcw_ref[...],
                             efcb_ref[...], beta_ref[...])
    hvn_ref[...] = hvn
    _write_T(T_ref, u, w)


def _k0(x, nfcw, nfcb, efcw, efcb, lng, lnb, beta):
    full = lambda shp: pl.BlockSpec(shp, lambda i: (0,) * len(shp))
    return pl.pallas_call(
        _k0_body,
        grid=(NB,),
        in_specs=[
            pl.BlockSpec((BN, 256), lambda i: (i, 0)),
            full((256, H)), full((1, H)), full((1, H)), full((1, H)),
            full((1, H)), full((1, H)), full((1, 1)),
        ],
        out_specs=[
            pl.BlockSpec((BN, H), lambda i: (i, 0)),
            pl.BlockSpec((BN, H), lambda i: (i, 0)),
            pl.BlockSpec((4, BN, 32), lambda i: (0, i, 0)),
        ],
        out_shape=[
            jax.ShapeDtypeStruct((N, H), jnp.float32),
            jax.ShapeDtypeStruct((N, H), jnp.float32),
            jax.ShapeDtypeStruct((4, N, 32), jnp.float32),
        ],
    )(x, nfcw, nfcb, efcw, efcb, lng, lnb, beta)


def _klayer_body(prep, hvn_ref, acc_ref, hvp_ref, w1_ref, b1_ref, g1_ref,
                 bb1_ref, w2_ref, b2_ref, *rest):
    if prep:
        (efcw_ref, efcb_ref, lng_ref, lnb_ref, beta_ref,
         hv_ref, hvn2_ref, T_ref) = rest
    else:
        (hv_ref,) = rest
    numer = jnp.concatenate([acc_ref[0, :, :], acc_ref[1, :, :]], axis=-1)
    denom = jnp.concatenate([acc_ref[2, :, :], acc_ref[3, :, :]], axis=-1)
    good = denom > 0
    msg = jnp.where(good, numer / jnp.where(good, denom, 1.0), 0.0)
    feats = hvn_ref[...] + msg
    h = jnp.dot(feats, w1_ref[...], preferred_element_type=jnp.float32)
    h = jax.nn.relu(_ln(h + b1_ref[...], g1_ref[...], bb1_ref[...]))
    hv = jnp.dot(h, w2_ref[...], preferred_element_type=jnp.float32)
    hv = hv + b2_ref[...] + hvp_ref[...]
    hv_ref[...] = hv
    if prep:
        hvn, u, w = _prep_tables(hv, lng_ref[...], lnb_ref[...], efcw_ref[...],
                                 efcb_ref[...], beta_ref[...])
        hvn2_ref[...] = hvn
        _write_T(T_ref, u, w)


def _klayer(prep, hvn, acc, hvp, w1, b1, g1, bb1, w2, b2, *extra):
    full = lambda shp: pl.BlockSpec(shp, lambda i: (0,) * len(shp))
    rowspec = pl.BlockSpec((BN, H), lambda i: (i, 0))
    in_specs = [
        rowspec,
        pl.BlockSpec((4, BN, 32), lambda i: (0, i, 0)),
        rowspec,
        full((H, 2 * H)), full((1, 2 * H)), full((1, 2 * H)),
        full((1, 2 * H)), full((2 * H, H)), full((1, H)),
    ]
    out_specs = [rowspec]
    out_shape = [jax.ShapeDtypeStruct((N, H), jnp.float32)]
    if prep:
        in_specs += [full((1, H)), full((1, H)), full((1, H)), full((1, H)),
                     full((1, 1))]
        out_specs += [rowspec, pl.BlockSpec((4, BN, 32), lambda i: (0, i, 0))]
        out_shape += [jax.ShapeDtypeStruct((N, H), jnp.float32),
                      jax.ShapeDtypeStruct((4, N, 32), jnp.float32)]
    return pl.pallas_call(
        functools.partial(_klayer_body, prep),
        grid=(NB,),
        in_specs=in_specs,
        out_specs=out_specs,
        out_shape=out_shape,
    )(hvn, acc, hvp, w1, b1, g1, bb1, w2, b2, *extra)


def _k4_body(hv0_ref, hv1_ref, hv2_ref, hv3_ref, phiw_ref, phib_ref, wa_ref,
             ba_ref, wb_ref, bbat_ref, wc_ref, bc_ref, rhow_ref, rhob_ref,
             clsw_ref, clsb_ref, hpath_ref, out_ref, outfeat_ref,
             acch_ref, acce_ref):
    i = pl.program_id(0)
    xcat = jnp.concatenate(
        [hv0_ref[...], hv1_ref[...], hv2_ref[...], hv3_ref[...]], axis=-1)
    hp = jnp.dot(xcat, phiw_ref[...], preferred_element_type=jnp.float32)
    hp = jax.nn.relu(hp + phib_ref[...])
    hpath_ref[...] = hp
    a = jnp.tanh(jnp.dot(hp, wa_ref[...], preferred_element_type=jnp.float32)
                 + ba_ref[...])
    b = jax.nn.sigmoid(
        jnp.dot(hp, wb_ref[...], preferred_element_type=jnp.float32)
        + bbat_ref[...])
    gate = jnp.dot(a * b, wc_ref[...], preferred_element_type=jnp.float32)
    gate = gate + bc_ref[...]
    shift = jnp.sum(jnp.abs(wc_ref[...])) + jnp.abs(bc_ref[0, 0])
    e = jnp.exp(gate - shift)
    se = jnp.sum(e)
    seh = jnp.sum(e * hp, axis=0, keepdims=True)

    @pl.when(i == 0)
    def _():
        acch_ref[...] = seh
        acce_ref[...] = jnp.full((1, 256), se, jnp.float32)

    @pl.when(i > 0)
    def _():
        acch_ref[...] += seh
        acce_ref[...] += jnp.full((1, 256), se, jnp.float32)

    @pl.when(i == NB - 1)
    def _():
        hg = acch_ref[...] / acce_ref[...]
        of = jnp.dot(hg, rhow_ref[...], preferred_element_type=jnp.float32)
        of = jax.nn.relu(of + rhob_ref[...])
        outfeat_ref[...] = of
        out_ref[...] = jnp.dot(of, clsw_ref[...],
                               preferred_element_type=jnp.float32) + clsb_ref[...]


def _k4(hv0, hv1, hv2, hv3, phiw, phib, wa, ba, wb, bbat, wc, bc, rhow, rhob,
        clsw, clsb):
    full = lambda shp: pl.BlockSpec(shp, lambda i: (0,) * len(shp))
    rowspec = pl.BlockSpec((BN, H), lambda i: (i, 0))
    D = 4 * H
    return pl.pallas_call(
        _k4_body,
        grid=(NB,),
        in_specs=[
            rowspec, rowspec, rowspec, rowspec,
            full((D, D)), full((1, D)), full((D, D)), full((1, D)),
            full((D, D)), full((1, D)), full((D, 1)), full((1, 1)),
            full((D, H)), full((1, H)), full((H, 2)), full((1, 2)),
        ],
        out_specs=[
            pl.BlockSpec((BN, D), lambda i: (i, 0)),
            full((1, 2)), full((1, H)),
        ],
        out_shape=[
            jax.ShapeDtypeStruct((N, D), jnp.float32),
            jax.ShapeDtypeStruct((1, 2), jnp.float32),
            jax.ShapeDtypeStruct((1, H), jnp.float32),
        ],
        scratch_shapes=[
            pltpu.VMEM((1, D), jnp.float32),
            pltpu.VMEM((1, D), jnp.float32),
        ],
    )(hv0, hv1, hv2, hv3, phiw, phib, wa, ba, wb, bbat, wc, bc, rhow, rhob,
      clsw, clsb)


def _sc_body(T_ref, src_ref, dst_ref, zr_ref, out_ref,
             sacc, isA, idA, isB, idB, rA, rB, zbuf, semA, semB):
    c = lax.axis_index("c")
    s = lax.axis_index("s")
    pltpu.sync_copy(zr_ref, zbuf)

    fl0 = s * FPT

    def zero_own_range():
        def zstep(j, _):
            pltpu.sync_copy(zbuf, sacc.at[pl.ds(fl0 + j * 128, 128)])
            return 0
        lax.fori_loop(0, FPT // 128, zstep, 0)
        pltpu.sync_copy(zbuf.at[pl.ds(0, FPT - (FPT // 128) * 128)],
                        sacc.at[pl.ds(fl0 + (FPT // 128) * 128,
                                      FPT - (FPT // 128) * 128)])

    r_base = s * RPT
    for sl in range(2):
        slab = 2 * c + sl
        zero_own_range()
        plsc.subcore_barrier()

        def load_idx(b, isb, idb):
            r0 = r_base + b * NR
            pltpu.sync_copy(src_ref.at[slab, pl.ds(r0, NR)], isb)
            pltpu.sync_copy(dst_ref.at[pl.ds(r0, NR)], idb)

        def fire(isb, rbuf, sem):
            for j in range(NR):
                pltpu.async_copy(T_ref.at[isb.at[j]], rbuf.at[j], sem)

        def drain(isb, rbuf, sem):
            for j in range(NR):
                pltpu.make_async_copy(T_ref.at[isb.at[j]], rbuf.at[j],
                                      sem).wait()

        def scat(idb, rbuf):
            for j in range(NR):
                pltpu.sync_copy(rbuf.at[j], sacc.at[idb.at[j]], add=True)

        load_idx(0, isA, idA)
        fire(isA, rA, semA)

        def pstep(i, _):
            b1 = 2 * i + 1
            load_idx(b1, isB, idB)
            fire(isB, rB, semB)
            drain(isA, rA, semA)
            scat(idA, rA)

            @pl.when(b1 + 1 < NBLK)
            def _():
                load_idx(b1 + 1, isA, idA)
                fire(isA, rA, semA)
            drain(isB, rB, semB)
            scat(idB, rB)
            return 0
        lax.fori_loop(0, NBLK // 2, pstep, 0)

        plsc.subcore_barrier()
        pltpu.sync_copy(sacc.at[pl.ds(fl0, FPT)],
                        out_ref.at[pl.ds(slab * NF + fl0, FPT)])


@functools.partial(
    pl.kernel,
    out_type=jax.ShapeDtypeStruct((4 * NF, 32), jnp.float32),
    mesh=plsc.VectorSubcoreMesh(core_axis_name="c", subcore_axis_name="s"),
    scratch_types=[
        pltpu.VMEM_SHARED((NACC, 32), jnp.float32),
        pltpu.VMEM((NR, LANE), jnp.int32),
        pltpu.VMEM((NR, LANE), jnp.int32),
        pltpu.VMEM((NR, LANE), jnp.int32),
        pltpu.VMEM((NR, LANE), jnp.int32),
        pltpu.VMEM((NR, LANE, 32), jnp.float32),
        pltpu.VMEM((NR, LANE, 32), jnp.float32),
        pltpu.VMEM((LANE, 32), jnp.float32),
        pltpu.SemaphoreType.DMA,
        pltpu.SemaphoreType.DMA,
    ],
    compiler_params=pltpu.CompilerParams(use_tc_tiling_on_sc=False),
)
def _sc_scatter(T_ref, src_ref, dst_ref, zr_ref, out_ref,
                sacc, isA, idA, isB, idB, rA, rB, zbuf, semA, semB):
    _sc_body(T_ref, src_ref, dst_ref, zr_ref, out_ref,
             sacc, isA, idA, isB, idB, rA, rB, zbuf, semA, semB)


def kernel(x, edge_index, nfc_w, nfc_b, efc_w, efc_b, ln_g, ln_b, betas,
           w1, b1, g1, bb1, w2, b2, phi_w, phi_b, wa, ba, wb, bb_attn,
           wc, bc, rho_w, rho_b, cls_w, cls_b):
    f32 = jnp.float32
    src = edge_index[0].astype(jnp.int32)
    dst = edge_index[1].astype(jnp.int32)
    pad = E_PAD - E
    srcp = jnp.concatenate([src, jnp.zeros((pad,), jnp.int32)])
    dstp = jnp.concatenate([dst, jnp.full((pad,), N, jnp.int32)])
    src4 = jnp.stack([srcp + s * N for s in range(4)]).reshape(4, E_ROWS, LANE)
    dst2 = dstp.reshape(E_ROWS, LANE)
    zrows = jnp.zeros((LANE, 32), f32)

    r2 = lambda v: v.reshape(1, -1)
    hv0, hvn, T = _k0(x, nfc_w, r2(nfc_b), efc_w, r2(efc_b),
                      r2(ln_g[0]), r2(ln_b[0]), betas[0].reshape(1, 1))
    hvs = [hv0]
    for l in range(L):
        acc = _sc_scatter(T.reshape(4 * N, 32), src4, dst2, zrows)
        acc = acc.reshape(4, NF, 32)
        prep = l < L - 1
        extra = ()
        if prep:
            extra = (efc_w, r2(efc_b), r2(ln_g[l + 1]), r2(ln_b[l + 1]),
                     betas[l + 1].reshape(1, 1))
        res = _klayer(prep, hvn, acc, hvs[-1], w1[l], r2(b1[l]), r2(g1[l]),
                      r2(bb1[l]), w2[l], r2(b2[l]), *extra)
        if prep:
            hv, hvn, T = res
        else:
            (hv,) = res
        hvs.append(hv)

    h_path, out, out_feat = _k4(hvs[0], hvs[1], hvs[2], hvs[3], phi_w,
                                r2(phi_b), wa, r2(ba), wb, r2(bb_attn), wc,
                                bc.reshape(1, 1), rho_w, r2(rho_b), cls_w,
                                r2(cls_b))
    return (out, out_feat, h_path)


# speedup vs baseline: 10.0014x; 1.1458x over previous
import functools

import jax
import jax.numpy as jnp
from jax import lax
from jax.experimental import pallas as pl
from jax.experimental.pallas import tpu as pltpu
from jax.experimental.pallas import tpu_sc as plsc

N = 50000
E = 800000
H = 64
L = 3
EPS = 1e-07

BN = 2000
NB = N // BN

_NS = 16
LANE = 128
RPT = 400
E_ROWS = RPT * _NS
E_PAD = E_ROWS * LANE
RING = 5
LOOK = RING - 1
CH = 40
NCH = RPT // CH
FPT = 3128
NF = FPT * _NS
NACC = NF


def _ln(x, g, b, eps=1e-5):
    mu = jnp.mean(x, axis=-1, keepdims=True)
    var = jnp.var(x, axis=-1, keepdims=True)
    return (x - mu) * jax.lax.rsqrt(var + eps) * g + b


def _prep_tables(hv, lng, lnb, efcw, efcb, beta):
    hvn = jax.nn.relu(_ln(hv, lng, lnb))
    c = jax.nn.relu(efcw + efcb)
    p = jax.nn.relu(hvn + c) + EPS
    w = jnp.exp(beta * p)
    u = p * w
    return hvn, u, w


def _write_T(T_ref, u, w):
    T_ref[0, :, :] = u[:, 0:32]
    T_ref[1, :, :] = u[:, 32:64]
    T_ref[2, :, :] = w[:, 0:32]
    T_ref[3, :, :] = w[:, 32:64]


def _k0_body(x_ref, nfcw_ref, nfcb_ref, efcw_ref, efcb_ref, lng_ref, lnb_ref,
             beta_ref, hv0_ref, hvn_ref, T_ref):
    hv = jnp.dot(x_ref[...], nfcw_ref[...], preferred_element_type=jnp.float32)
    hv = jax.nn.relu(hv + nfcb_ref[...])
    hv0_ref[...] = hv
    hvn, u, w = _prep_tables(hv, lng_ref[...], lnb_ref[...], efcw_ref[...],
                             efcb_ref[...], beta_ref[...])
    hvn_ref[...] = hvn
    _write_T(T_ref, u, w)


def _k0(x, nfcw, nfcb, efcw, efcb, lng, lnb, beta):
    full = lambda shp: pl.BlockSpec(shp, lambda i: (0,) * len(shp))
    return pl.pallas_call(
        _k0_body,
        grid=(NB,),
        in_specs=[
            pl.BlockSpec((BN, 256), lambda i: (i, 0)),
            full((256, H)), full((1, H)), full((1, H)), full((1, H)),
            full((1, H)), full((1, H)), full((1, 1)),
        ],
        out_specs=[
            pl.BlockSpec((BN, H), lambda i: (i, 0)),
            pl.BlockSpec((BN, H), lambda i: (i, 0)),
            pl.BlockSpec((4, BN, 32), lambda i: (0, i, 0)),
        ],
        out_shape=[
            jax.ShapeDtypeStruct((N, H), jnp.float32),
            jax.ShapeDtypeStruct((N, H), jnp.float32),
            jax.ShapeDtypeStruct((4, N, 32), jnp.float32),
        ],
    )(x, nfcw, nfcb, efcw, efcb, lng, lnb, beta)


def _klayer_body(prep, hvn_ref, acc_ref, hvp_ref, w1_ref, b1_ref, g1_ref,
                 bb1_ref, w2_ref, b2_ref, *rest):
    if prep:
        (efcw_ref, efcb_ref, lng_ref, lnb_ref, beta_ref,
         hv_ref, hvn2_ref, T_ref) = rest
    else:
        (hv_ref,) = rest
    numer = jnp.concatenate([acc_ref[0, :, :], acc_ref[1, :, :]], axis=-1)
    denom = jnp.concatenate([acc_ref[2, :, :], acc_ref[3, :, :]], axis=-1)
    good = denom > 0
    msg = jnp.where(good, numer / jnp.where(good, denom, 1.0), 0.0)
    feats = hvn_ref[...] + msg
    h = jnp.dot(feats, w1_ref[...], preferred_element_type=jnp.float32)
    h = jax.nn.relu(_ln(h + b1_ref[...], g1_ref[...], bb1_ref[...]))
    hv = jnp.dot(h, w2_ref[...], preferred_element_type=jnp.float32)
    hv = hv + b2_ref[...] + hvp_ref[...]
    hv_ref[...] = hv
    if prep:
        hvn, u, w = _prep_tables(hv, lng_ref[...], lnb_ref[...], efcw_ref[...],
                                 efcb_ref[...], beta_ref[...])
        hvn2_ref[...] = hvn
        _write_T(T_ref, u, w)


def _klayer(prep, hvn, acc, hvp, w1, b1, g1, bb1, w2, b2, *extra):
    full = lambda shp: pl.BlockSpec(shp, lambda i: (0,) * len(shp))
    rowspec = pl.BlockSpec((BN, H), lambda i: (i, 0))
    in_specs = [
        rowspec,
        pl.BlockSpec((4, BN, 32), lambda i: (0, i, 0)),
        rowspec,
        full((H, 2 * H)), full((1, 2 * H)), full((1, 2 * H)),
        full((1, 2 * H)), full((2 * H, H)), full((1, H)),
    ]
    out_specs = [rowspec]
    out_shape = [jax.ShapeDtypeStruct((N, H), jnp.float32)]
    if prep:
        in_specs += [full((1, H)), full((1, H)), full((1, H)), full((1, H)),
                     full((1, 1))]
        out_specs += [rowspec, pl.BlockSpec((4, BN, 32), lambda i: (0, i, 0))]
        out_shape += [jax.ShapeDtypeStruct((N, H), jnp.float32),
                      jax.ShapeDtypeStruct((4, N, 32), jnp.float32)]
    return pl.pallas_call(
        functools.partial(_klayer_body, prep),
        grid=(NB,),
        in_specs=in_specs,
        out_specs=out_specs,
        out_shape=out_shape,
    )(hvn, acc, hvp, w1, b1, g1, bb1, w2, b2, *extra)


def _k4_body(hv0_ref, hv1_ref, hv2_ref, hv3_ref, phiw_ref, phib_ref, wa_ref,
             ba_ref, wb_ref, bbat_ref, wc_ref, bc_ref, rhow_ref, rhob_ref,
             clsw_ref, clsb_ref, hpath_ref, out_ref, outfeat_ref,
             acch_ref, acce_ref):
    i = pl.program_id(0)
    xcat = jnp.concatenate(
        [hv0_ref[...], hv1_ref[...], hv2_ref[...], hv3_ref[...]], axis=-1)
    hp = jnp.dot(xcat, phiw_ref[...], preferred_element_type=jnp.float32)
    hp = jax.nn.relu(hp + phib_ref[...])
    hpath_ref[...] = hp
    a = jnp.tanh(jnp.dot(hp, wa_ref[...], preferred_element_type=jnp.float32)
                 + ba_ref[...])
    b = jax.nn.sigmoid(
        jnp.dot(hp, wb_ref[...], preferred_element_type=jnp.float32)
        + bbat_ref[...])
    gate = jnp.dot(a * b, wc_ref[...], preferred_element_type=jnp.float32)
    gate = gate + bc_ref[...]
    shift = jnp.sum(jnp.abs(wc_ref[...])) + jnp.abs(bc_ref[0, 0])
    e = jnp.exp(gate - shift)
    se = jnp.sum(e)
    seh = jnp.sum(e * hp, axis=0, keepdims=True)

    @pl.when(i == 0)
    def _():
        acch_ref[...] = seh
        acce_ref[...] = jnp.full((1, 256), se, jnp.float32)

    @pl.when(i > 0)
    def _():
        acch_ref[...] += seh
        acce_ref[...] += jnp.full((1, 256), se, jnp.float32)

    @pl.when(i == NB - 1)
    def _():
        hg = acch_ref[...] / acce_ref[...]
        of = jnp.dot(hg, rhow_ref[...], preferred_element_type=jnp.float32)
        of = jax.nn.relu(of + rhob_ref[...])
        outfeat_ref[...] = of
        out_ref[...] = jnp.dot(of, clsw_ref[...],
                               preferred_element_type=jnp.float32) + clsb_ref[...]


def _k4(hv0, hv1, hv2, hv3, phiw, phib, wa, ba, wb, bbat, wc, bc, rhow, rhob,
        clsw, clsb):
    full = lambda shp: pl.BlockSpec(shp, lambda i: (0,) * len(shp))
    rowspec = pl.BlockSpec((BN, H), lambda i: (i, 0))
    D = 4 * H
    return pl.pallas_call(
        _k4_body,
        grid=(NB,),
        in_specs=[
            rowspec, rowspec, rowspec, rowspec,
            full((D, D)), full((1, D)), full((D, D)), full((1, D)),
            full((D, D)), full((1, D)), full((D, 1)), full((1, 1)),
            full((D, H)), full((1, H)), full((H, 2)), full((1, 2)),
        ],
        out_specs=[
            pl.BlockSpec((BN, D), lambda i: (i, 0)),
            full((1, 2)), full((1, H)),
        ],
        out_shape=[
            jax.ShapeDtypeStruct((N, D), jnp.float32),
            jax.ShapeDtypeStruct((1, 2), jnp.float32),
            jax.ShapeDtypeStruct((1, H), jnp.float32),
        ],
        scratch_shapes=[
            pltpu.VMEM((1, D), jnp.float32),
            pltpu.VMEM((1, D), jnp.float32),
        ],
    )(hv0, hv1, hv2, hv3, phiw, phib, wa, ba, wb, bbat, wc, bc, rhow, rhob,
      clsw, clsb)


def _sc_body(T_ref, src_ref, dst_ref, zr_ref, out_ref,
             sacc, isb, idb, rbufs, gsems, ssems):
    c = lax.axis_index("c")
    s = lax.axis_index("s")
    fl0 = s * FPT
    r_base = s * RPT

    def zero_own_range():
        pltpu.sync_copy(zr_ref, rbufs[0])

        def zstep(j, _):
            pltpu.sync_copy(rbufs[0], sacc.at[pl.ds(fl0 + j * 128, 128)])
            return 0
        lax.fori_loop(0, FPT // 128, zstep, 0)
        pltpu.sync_copy(rbufs[0].at[pl.ds(0, FPT - (FPT // 128) * 128)],
                        sacc.at[pl.ds(fl0 + (FPT // 128) * 128,
                                      FPT - (FPT // 128) * 128)])

    def fire_g(r, k):
        pltpu.async_copy(T_ref.at[isb.at[r]], rbufs[k], gsems[k])

    def drain_g(r, k):
        pltpu.make_async_copy(T_ref.at[isb.at[r]], rbufs[k], gsems[k]).wait()

    def fire_s(r, k):
        pltpu.async_copy(rbufs[k], sacc.at[idb.at[r]], ssems[k], add=True)

    def drain_s(r, k):
        pltpu.make_async_copy(rbufs[k], sacc.at[idb.at[r]], ssems[k]).wait()

    for sl in range(2):
        slab = 2 * c + sl
        zero_own_range()
        plsc.subcore_barrier()

        def chunk(ci, _):
            r0 = r_base + ci * CH
            pltpu.sync_copy(src_ref.at[slab, pl.ds(r0, CH)], isb)
            pltpu.sync_copy(dst_ref.at[pl.ds(r0, CH)], idb)
            for k in range(LOOK):
                fire_g(k, k)

            def group(g, _):
                for k in range(RING):
                    j = g * RING + k
                    drain_g(j, k)
                    fire_s(j, k)
                    kn = (k + LOOK) % RING

                    @pl.when(j + LOOK < CH)
                    def _():
                        @pl.when(j >= 1)
                        def _():
                            drain_s(j - 1, kn)
                        fire_g(j + LOOK, kn)
                return 0
            lax.fori_loop(0, CH // RING, group, 0)
            for k in range(RING):
                drain_s(CH - RING + k, k)
            return 0
        lax.fori_loop(0, NCH, chunk, 0)

        plsc.subcore_barrier()
        pltpu.sync_copy(sacc.at[pl.ds(fl0, FPT)],
                        out_ref.at[pl.ds(slab * NF + fl0, FPT)])


@functools.partial(
    pl.kernel,
    out_type=jax.ShapeDtypeStruct((4 * NF, 32), jnp.float32),
    mesh=plsc.VectorSubcoreMesh(core_axis_name="c", subcore_axis_name="s"),
    scratch_types=(
        [pltpu.VMEM_SHARED((NACC, 32), jnp.float32),
         pltpu.VMEM((CH, LANE), jnp.int32),
         pltpu.VMEM((CH, LANE), jnp.int32)]
        + [pltpu.VMEM((LANE, 32), jnp.float32)] * RING
        + [pltpu.SemaphoreType.DMA] * (2 * RING)
    ),
    compiler_params=pltpu.CompilerParams(use_tc_tiling_on_sc=False),
)
def _sc_scatter(T_ref, src_ref, dst_ref, zr_ref, out_ref,
                sacc, isb, idb, *rest):
    rbufs = rest[:RING]
    gsems = rest[RING:2 * RING]
    ssems = rest[2 * RING:3 * RING]
    _sc_body(T_ref, src_ref, dst_ref, zr_ref, out_ref,
             sacc, isb, idb, rbufs, gsems, ssems)


def kernel(x, edge_index, nfc_w, nfc_b, efc_w, efc_b, ln_g, ln_b, betas,
           w1, b1, g1, bb1, w2, b2, phi_w, phi_b, wa, ba, wb, bb_attn,
           wc, bc, rho_w, rho_b, cls_w, cls_b):
    f32 = jnp.float32
    src = edge_index[0].astype(jnp.int32)
    dst = edge_index[1].astype(jnp.int32)
    pad = E_PAD - E
    srcp = jnp.concatenate([src, jnp.zeros((pad,), jnp.int32)])
    dstp = jnp.concatenate([dst, jnp.full((pad,), N, jnp.int32)])
    src4 = jnp.stack([srcp + s * N for s in range(4)]).reshape(4, E_ROWS, LANE)
    dst2 = dstp.reshape(E_ROWS, LANE)
    zrows = jnp.zeros((LANE, 32), f32)

    r2 = lambda v: v.reshape(1, -1)
    hv0, hvn, T = _k0(x, nfc_w, r2(nfc_b), efc_w, r2(efc_b),
                      r2(ln_g[0]), r2(ln_b[0]), betas[0].reshape(1, 1))
    hvs = [hv0]
    for l in range(L):
        acc = _sc_scatter(T.reshape(4 * N, 32), src4, dst2, zrows)
        acc = acc.reshape(4, NF, 32)
        prep = l < L - 1
        extra = ()
        if prep:
            extra = (efc_w, r2(efc_b), r2(ln_g[l + 1]), r2(ln_b[l + 1]),
                     betas[l + 1].reshape(1, 1))
        res = _klayer(prep, hvn, acc, hvs[-1], w1[l], r2(b1[l]), r2(g1[l]),
                      r2(bb1[l]), w2[l], r2(b2[l]), *extra)
        if prep:
            hv, hvn, T = res
        else:
            (hv,) = res
        hvs.append(hv)

    h_path, out, out_feat = _k4(hvs[0], hvs[1], hvs[2], hvs[3], phi_w,
                                r2(phi_b), wa, r2(ba), wb, r2(bb_attn), wc,
                                bc.reshape(1, 1), rho_w, r2(rho_b), cls_w,
                                r2(cls_b))
    return (out, out_feat, h_path)
